# baseline (speedup 1.0000x reference)
import functools

import jax
import jax.numpy as jnp
from jax import lax
from jax.experimental import pallas as pl
from jax.experimental.pallas import tpu as pltpu
from jax.experimental.pallas import tpu_sc as plsc

N_FIELDS = 26
VOCAB = 100000
EMBED_D = 32
BATCH = 16384
NUM_CORES = 2
NUM_SUBCORES = 16
NUM_WORKERS = NUM_CORES * NUM_SUBCORES
LANES = 16

ROWS_W = BATCH // NUM_WORKERS
GW = 4 * N_FIELDS
NG = ROWS_W // 4
ROW_PAD = 112
NBUF = 2


def _tree_sum(vals):
    while len(vals) > 1:
        nxt = [vals[i] + vals[i + 1] for i in range(0, len(vals) - 1, 2)]
        if len(vals) % 2:
            nxt.append(vals[-1])
        vals = nxt
    return vals[0]


def _body(inp_hbm, table_hbm, out_hbm, idx_v, pat_v, acc_v, buf_v,
          sem_in, sem0, sem1):
    wid = lax.axis_index("s") * NUM_CORES + lax.axis_index("c")
    sems = (sem0, sem1)

    in_cp = pltpu.async_copy(
        inp_hbm.at[pl.ds(wid * NG, NG), :], idx_v.at[:, pl.ds(0, GW)], sem_in)

    iota = lax.iota(jnp.int32, LANES)
    for h in range(ROW_PAD // LANES):
        pat_v[h, :] = ((h * LANES + iota) % N_FIELDS) * VOCAB
    in_cp.wait()

    def adjust(k):
        for h in range(ROW_PAD // LANES):
            sl = pl.ds(h * LANES, LANES)
            idx_v[k, sl] = idx_v[k, sl] + pat_v[h, :]

    def start(k, b):
        pltpu.async_copy(
            table_hbm.at[idx_v.at[k, pl.ds(0, GW)]], buf_v.at[b], sems[b])

    adjust(0)
    adjust(1)
    start(0, 0)
    start(1, 1)

    def ring(g, carry):
        for b in range(NBUF):
            k = NBUF * g + b
            pltpu.make_async_copy(
                table_hbm.at[idx_v.at[k, pl.ds(0, GW)]], buf_v.at[b],
                sems[b]).wait()
            for br in range(4):
                arow = 4 * k + br
                for h in range(EMBED_D // LANES):
                    sl = pl.ds(h * LANES, LANES)
                    acc_v[arow, sl] = _tree_sum(
                        [buf_v[b, br * N_FIELDS + f, sl]
                         for f in range(N_FIELDS)])

            @pl.when(k + NBUF < NG)
            def _():
                adjust(k + NBUF)
                start(k + NBUF, b)
        return carry

    lax.fori_loop(0, NG // NBUF, ring, 0)
    pltpu.sync_copy(acc_v, out_hbm.at[pl.ds(wid * ROWS_W, ROWS_W), :])


@functools.partial(
    pl.kernel,
    out_type=jax.ShapeDtypeStruct((BATCH, EMBED_D), jnp.float32),
    mesh=plsc.VectorSubcoreMesh(core_axis_name="c", subcore_axis_name="s"),
    compiler_params=pltpu.CompilerParams(use_tc_tiling_on_sc=False),
    scratch_types=[
        pltpu.VMEM((NG, ROW_PAD), jnp.int32),
        pltpu.VMEM((ROW_PAD // LANES, LANES), jnp.int32),
        pltpu.VMEM((ROWS_W, EMBED_D), jnp.float32),
        pltpu.VMEM((NBUF, GW, EMBED_D), jnp.float32),
        pltpu.SemaphoreType.DMA,
        pltpu.SemaphoreType.DMA,
        pltpu.SemaphoreType.DMA,
    ],
)
def _attr_embed(inp_hbm, table_hbm, out_hbm, idx_v, pat_v, acc_v, buf_v,
                sem_in, sem0, sem1):
    _body(inp_hbm, table_hbm, out_hbm, idx_v, pat_v, acc_v, buf_v,
          sem_in, sem0, sem1)


def kernel(inputs, table):
    assert inputs.shape == (BATCH, N_FIELDS) and inputs.dtype == jnp.int32
    inp2 = inputs.reshape(NUM_WORKERS * NG, GW)
    return _attr_embed(inp2, table)
